# baseline (speedup 1.0000x reference)
import functools

import jax
import jax.numpy as jnp
from jax import lax
from jax.experimental import pallas as pl
from jax.experimental.pallas import tpu as pltpu
from jax.experimental.pallas import tpu_sc as plsc

NC = 2
NS = 16
NW = NC * NS
ROW_CHUNK = 512


def _make_sc_gather(n_ids: int, d: int):
    per_w = n_ids // NW
    n_chunks = per_w // ROW_CHUNK
    mesh = plsc.VectorSubcoreMesh(core_axis_name="c", subcore_axis_name="s")

    @functools.partial(
        pl.kernel,
        mesh=mesh,
        out_type=jax.ShapeDtypeStruct((n_ids, d), jnp.float32),
        scratch_types=[
            pltpu.VMEM((per_w,), jnp.int32),
            pltpu.VMEM((ROW_CHUNK, d), jnp.float32),
            pltpu.SemaphoreType.DMA,
            pltpu.SemaphoreType.DMA,
        ],
    )
    def gather_kernel(table_hbm, ids_hbm, out_hbm, ids_v, rows_v, sem, sem2):
        wid = lax.axis_index("s") * NC + lax.axis_index("c")
        base = wid * per_w
        pltpu.sync_copy(ids_hbm.at[pl.ds(base, per_w)], ids_v)

        def chunk_body(c, carry):

            def grp_body(g, carry2):
                v = ids_v[pl.ds(c * ROW_CHUNK + g * 16, 16)]
                for j in range(16):
                    pltpu.async_copy(
                        table_hbm.at[pl.ds(v[j], 1)],
                        rows_v.at[pl.ds(g * 16 + j, 1)],
                        sem if j % 2 == 0 else sem2,
                    )
                return carry2

            lax.fori_loop(0, ROW_CHUNK // 16, grp_body, 0)
            pltpu.make_async_copy(
                table_hbm.at[pl.ds(0, ROW_CHUNK // 2)],
                rows_v.at[pl.ds(0, ROW_CHUNK // 2)],
                sem,
            ).wait()
            pltpu.make_async_copy(
                table_hbm.at[pl.ds(0, ROW_CHUNK // 2)],
                rows_v.at[pl.ds(0, ROW_CHUNK // 2)],
                sem2,
            ).wait()
            pltpu.sync_copy(
                rows_v, out_hbm.at[pl.ds(base + c * ROW_CHUNK, ROW_CHUNK)]
            )
            return carry

        lax.fori_loop(0, n_chunks, chunk_body, 0)

    return gather_kernel


def _mlp_body(r_ref, w1_ref, b1_ref, w2_ref, b2_ref, out_ref):
    nk = r_ref.shape[0]
    d = w1_ref.shape[1]
    acc = jnp.dot(r_ref[0, :, :d], w1_ref[0],
                  preferred_element_type=jnp.float32)
    for k in range(1, nk):
        acc += jnp.dot(r_ref[k, :, :d], w1_ref[k],
                       preferred_element_type=jnp.float32)
    h = jnp.maximum(acc + b1_ref[...], 0.0)
    out_ref[...] = (
        jnp.sum(h * w2_ref[...], axis=1, keepdims=True) + b2_ref[...]
    )


def kernel(user_id, item_id, sub_node_id_list, value, table, W1, b1, W2, b2):
    del value
    b = user_id.shape[0]
    d = table.shape[1]
    n_sub = sub_node_id_list.shape[0]
    nk = n_sub + 2
    n_ids = nk * b

    slots = sub_node_id_list.reshape(-1).reshape(b, n_sub).T
    ids8 = jnp.concatenate(
        [user_id[None].astype(jnp.int32), item_id[None].astype(jnp.int32),
         slots.astype(jnp.int32)], axis=0
    )

    halves = 2
    bh = b // halves
    gather = _make_sc_gather(nk * bh, d)
    blk = 512
    w1r = W1.reshape(nk, d, d)
    b1r = b1.reshape(1, d)
    w2r = W2.reshape(1, d)
    b2r = b2.reshape(1, 1)
    outs = []
    for hh in range(halves):
        ids_h = ids8[:, hh * bh:(hh + 1) * bh].reshape(nk * bh)
        r = gather(table, ids_h)
        r3 = r.reshape(nk, bh, d)
        outs.append(pl.pallas_call(
            _mlp_body,
            grid=(bh // blk,),
            in_specs=[
                pl.BlockSpec((nk, blk, d), lambda i: (0, i, 0)),
                pl.BlockSpec((nk, d, d), lambda i: (0, 0, 0)),
                pl.BlockSpec((1, d), lambda i: (0, 0)),
                pl.BlockSpec((1, d), lambda i: (0, 0)),
                pl.BlockSpec((1, 1), lambda i: (0, 0)),
            ],
            out_specs=pl.BlockSpec((blk, 1), lambda i: (i, 0)),
            out_shape=jax.ShapeDtypeStruct((bh, 1), jnp.float32),
        )(r3, w1r, b1r, w2r, b2r))
    return jnp.concatenate(outs, axis=0)


# speedup vs baseline: 1.0090x; 1.0090x over previous
import functools

import jax
import jax.numpy as jnp
from jax import lax
from jax.experimental import pallas as pl
from jax.experimental.pallas import tpu as pltpu
from jax.experimental.pallas import tpu_sc as plsc

NC = 2
NS = 16
NW = NC * NS
ROW_CHUNK = 512


def _make_sc_gather(n_ids: int, d: int):
    per_w = n_ids // NW
    n_chunks = per_w // ROW_CHUNK
    mesh = plsc.VectorSubcoreMesh(core_axis_name="c", subcore_axis_name="s")

    @functools.partial(
        pl.kernel,
        mesh=mesh,
        out_type=jax.ShapeDtypeStruct((n_ids, d), jnp.float32),
        scratch_types=[
            pltpu.VMEM((per_w,), jnp.int32),
            pltpu.VMEM((ROW_CHUNK, d), jnp.float32),
            pltpu.SemaphoreType.DMA,
        ],
    )
    def gather_kernel(table_hbm, ids_hbm, out_hbm, ids_v, rows_v, sem):
        wid = lax.axis_index("s") * NC + lax.axis_index("c")
        base = wid * per_w
        pltpu.sync_copy(ids_hbm.at[pl.ds(base, per_w)], ids_v)

        def chunk_body(c, carry):

            def grp_body(g, carry2):
                v = ids_v[pl.ds(c * ROW_CHUNK + g * 16, 16)]
                for j in range(16):
                    pltpu.async_copy(
                        table_hbm.at[pl.ds(v[j], 1)],
                        rows_v.at[pl.ds(g * 16 + j, 1)],
                        sem,
                    )
                return carry2

            lax.fori_loop(0, ROW_CHUNK // 16, grp_body, 0)
            pltpu.make_async_copy(
                table_hbm.at[pl.ds(0, ROW_CHUNK)],
                rows_v,
                sem,
            ).wait()
            pltpu.sync_copy(
                rows_v, out_hbm.at[pl.ds(base + c * ROW_CHUNK, ROW_CHUNK)]
            )
            return carry

        lax.fori_loop(0, n_chunks, chunk_body, 0)

    return gather_kernel


def _mlp_body(r_ref, w1_ref, b1_ref, w2_ref, b2_ref, out_ref):
    nk = r_ref.shape[0]
    d = w1_ref.shape[1]
    acc = jnp.dot(r_ref[0, :, :d], w1_ref[0],
                  preferred_element_type=jnp.float32)
    for k in range(1, nk):
        acc += jnp.dot(r_ref[k, :, :d], w1_ref[k],
                       preferred_element_type=jnp.float32)
    h = jnp.maximum(acc + b1_ref[...], 0.0)
    out_ref[...] = (
        jnp.sum(h * w2_ref[...], axis=1, keepdims=True) + b2_ref[...]
    )


def kernel(user_id, item_id, sub_node_id_list, value, table, W1, b1, W2, b2):
    del value
    b = user_id.shape[0]
    d = table.shape[1]
    n_sub = sub_node_id_list.shape[0]
    nk = n_sub + 2
    n_ids = nk * b

    slots = sub_node_id_list.reshape(-1).reshape(b, n_sub).T
    ids = jnp.concatenate(
        [user_id[None].astype(jnp.int32), item_id[None].astype(jnp.int32),
         slots.astype(jnp.int32)], axis=0
    ).reshape(n_ids)

    r = _make_sc_gather(n_ids, d)(table, ids)
    r3 = r.reshape(nk, b, d)

    blk = 512
    grid = (b // blk,)
    out = pl.pallas_call(
        _mlp_body,
        grid=grid,
        in_specs=[
            pl.BlockSpec((nk, blk, d), lambda i: (0, i, 0)),
            pl.BlockSpec((nk, d, d), lambda i: (0, 0, 0)),
            pl.BlockSpec((1, d), lambda i: (0, 0)),
            pl.BlockSpec((1, d), lambda i: (0, 0)),
            pl.BlockSpec((1, 1), lambda i: (0, 0)),
        ],
        out_specs=pl.BlockSpec((blk, 1), lambda i: (i, 0)),
        out_shape=jax.ShapeDtypeStruct((b, 1), jnp.float32),
    )(r3, W1.reshape(nk, d, d), b1.reshape(1, d), W2.reshape(1, d),
      b2.reshape(1, 1))
    return out
